# baseline (speedup 1.0000x reference)
import functools
import math

import jax
import jax.numpy as jnp
from jax import lax
from jax.experimental import pallas as pl
from jax.experimental.pallas import tpu as pltpu
from jax.experimental.pallas import tpu_sc as plsc

N = 10000
D = 128
E = 320000
G = 100
L = 3

NC = 2
NS = 16
NW = NC * NS
EPW = E // NW
EB = 80
NB = EPW // EB
NPH = 5
HNB = NB // NPH
NBUF = 4
NPAD = 10240
RPT = NPAD // NS

_mesh = plsc.VectorSubcoreMesh(core_axis_name="c", subcore_axis_name="s")


@functools.partial(
    pl.kernel,
    out_type=[
        jax.ShapeDtypeStruct((NPAD,), jnp.float32),
        jax.ShapeDtypeStruct((NPAD,), jnp.float32),
    ],
    mesh=_mesh,
    scratch_types=[
        pltpu.VMEM_SHARED((NPAD,), jnp.float32),
        pltpu.VMEM((NB, 1, EB), jnp.int32),
        pltpu.VMEM((RPT,), jnp.float32),
        pltpu.VMEM((128,), jnp.float32),
    ],
)
def _deg_kernel(dst_hbm, o0_hbm, o1_hbm, acc, dst_v, zeros_v, ones_v):
    c = lax.axis_index("c")
    s = lax.axis_index("s")
    wid = c * NS + s
    for i in range(RPT // 16):
        zeros_v[pl.ds(i * 16, 16)] = jnp.zeros((16,), jnp.float32)
    for i in range(128 // 16):
        ones_v[pl.ds(i * 16, 16)] = jnp.ones((16,), jnp.float32)
    pltpu.sync_copy(zeros_v, acc.at[pl.ds(s * RPT, RPT)])
    pltpu.sync_copy(dst_hbm.at[wid], dst_v)
    plsc.subcore_barrier()

    @pl.loop(0, NB)
    def _(j):
        pltpu.sync_copy(ones_v.at[pl.ds(0, EB)], acc.at[dst_v.at[j, 0]],
                        add=True)

    plsc.subcore_barrier()

    @pl.when(c == 0)
    def _():
        pltpu.sync_copy(acc.at[pl.ds(s * RPT, RPT)],
                        o0_hbm.at[pl.ds(s * RPT, RPT)])

    @pl.when(c == 1)
    def _():
        pltpu.sync_copy(acc.at[pl.ds(s * RPT, RPT)],
                        o1_hbm.at[pl.ds(s * RPT, RPT)])


@functools.partial(
    pl.kernel,
    out_type=jax.ShapeDtypeStruct((NC, NPAD, D), jnp.float32),
    mesh=_mesh,
    scratch_types=[
        pltpu.VMEM_SHARED((NPAD, D), jnp.float32),
        pltpu.VMEM((HNB, 1, EB), jnp.int32),
        pltpu.VMEM((HNB, 1, EB), jnp.int32),
        pltpu.VMEM((NBUF, EB, D), jnp.float32),
    ] + [pltpu.SemaphoreType.DMA] * NBUF,
)
def _agg_kernel(hp_hbm, src_hbm, dst_hbm, out_hbm, acc, src_v, dst_v, buf,
                *gsems):
    c = lax.axis_index("c")
    s = lax.axis_index("s")
    wid = c * NS + s
    pltpu.sync_copy(hp_hbm.at[pl.ds(s * RPT, RPT)],
                    acc.at[pl.ds(s * RPT, RPT)])
    plsc.subcore_barrier()

    @pl.loop(0, NPH)
    def _(p):
        pltpu.sync_copy(src_hbm.at[wid, pl.ds(p * HNB, HNB)], src_v)
        pltpu.sync_copy(dst_hbm.at[wid, pl.ds(p * HNB, HNB)], dst_v)
        for b in range(NBUF):
            pltpu.async_copy(hp_hbm.at[src_v.at[b, 0]], buf.at[b], gsems[b])

        @pl.loop(0, HNB - 1, step=NBUF)
        def _(j):
            for b in range(NBUF):
                pltpu.make_async_copy(hp_hbm.at[src_v.at[0, 0]], buf.at[b],
                                      gsems[b]).wait()
                pltpu.sync_copy(buf.at[b], acc.at[dst_v.at[j + b, 0]],
                                add=True)

                @pl.when(j + NBUF + b < HNB)
                def _(b=b):
                    pltpu.async_copy(hp_hbm.at[src_v.at[j + NBUF + b, 0]],
                                     buf.at[b], gsems[b])

        pltpu.make_async_copy(hp_hbm.at[src_v.at[0, 0]], buf.at[0],
                              gsems[0]).wait()
        pltpu.sync_copy(buf.at[0], acc.at[dst_v.at[HNB - 1, 0]], add=True)

    plsc.subcore_barrier()
    pltpu.sync_copy(acc.at[pl.ds(s * RPT, RPT)],
                    out_hbm.at[c, pl.ds(s * RPT, RPT)])


def _proj_body(deg1_ref, h_ref, w_ref, o_ref):
    dinv = lax.rsqrt(deg1_ref[...])
    o_ref[...] = jnp.dot(h_ref[...], w_ref[...],
                         preferred_element_type=jnp.float32) * dinv


_proj = pl.pallas_call(
    _proj_body,
    out_shape=jax.ShapeDtypeStruct((NPAD, D), jnp.float32),
)


def _readout_body(deg1_ref, p0_ref, p1_ref, hp_ref, b_ref, wk_ref, bk_ref,
                  wq_ref, bq_ref, s_ref, st_ref, out_ref, r_ref):
    dinv = lax.rsqrt(deg1_ref[...])
    out = (p0_ref[...] + p1_ref[...] - hp_ref[...]) * dinv + b_ref[...]
    out_ref[...] = out
    k = jnp.dot(out, wk_ref[...], preferred_element_type=jnp.float32) + bk_ref[...]
    m = jnp.dot(s_ref[...], out, preferred_element_type=jnp.float32) * (1.0 / (N // G))
    q = jnp.dot(m, wq_ref[...], preferred_element_type=jnp.float32) + bq_ref[...]
    qb = jnp.dot(st_ref[...], q, preferred_element_type=jnp.float32)
    t = jnp.sum(k * qb, axis=1, keepdims=True) * (1.0 / math.sqrt(D))
    att = jax.nn.sigmoid(t)
    r_ref[...] = jnp.dot(s_ref[...], out * att,
                         preferred_element_type=jnp.float32) * (1.0 / (N // G))


_readout = pl.pallas_call(
    _readout_body,
    out_shape=[
        jax.ShapeDtypeStruct((NPAD, D), jnp.float32),
        jax.ShapeDtypeStruct((G, D), jnp.float32),
    ],
)


def kernel(x, edge_index, batch, params):
    src = edge_index[0].reshape(NW, NB, 1, EB)
    dst = edge_index[1].reshape(NW, NB, 1, EB)
    d0, d1 = _deg_kernel(dst)
    deg1 = (d0 + d1 + 1.0).reshape(NPAD, 1)
    xp = jnp.concatenate(
        [x, jnp.zeros((NPAD - N, D), jnp.float32)], axis=0)
    batchp = jnp.concatenate(
        [batch, jnp.full((NPAD - N,), -1, batch.dtype)])
    gids = jnp.arange(G, dtype=batch.dtype)
    s_mat = (batchp[None, :] == gids[:, None]).astype(jnp.float32)
    st_mat = (batchp[:, None] == gids[None, :]).astype(jnp.float32)
    h = xp
    outs = []
    for i in range(L):
        w = params['W%d' % i]
        b = params['b%d' % i].reshape(1, D)
        wq = params['Wq%d' % i]
        bq = params['bq%d' % i].reshape(1, D)
        wk = params['Wk%d' % i]
        bk = params['bk%d' % i].reshape(1, D)
        hp = _proj(deg1, h, w)
        parts = _agg_kernel(hp, src, dst)
        h, r = _readout(deg1, parts[0], parts[1], hp, b, wk, bk, wq, bq,
                        s_mat, st_mat)
        outs.append(r)
    return jnp.concatenate(outs, axis=1)


# speedup vs baseline: 1.0193x; 1.0193x over previous
import functools
import math

import jax
import jax.numpy as jnp
from jax import lax
from jax.experimental import pallas as pl
from jax.experimental.pallas import tpu as pltpu
from jax.experimental.pallas import tpu_sc as plsc

N = 10000
D = 128
E = 320000
G = 100
L = 3

NC = 2
NS = 16
NW = NC * NS
EPW = E // NW
EB = 100
NB = EPW // EB
NPH = 4
HNB = NB // NPH
NBUF = 3
NPAD = 10240
RPT = NPAD // NS

_mesh = plsc.VectorSubcoreMesh(core_axis_name="c", subcore_axis_name="s")


@functools.partial(
    pl.kernel,
    out_type=[
        jax.ShapeDtypeStruct((NPAD,), jnp.float32),
        jax.ShapeDtypeStruct((NPAD,), jnp.float32),
    ],
    mesh=_mesh,
    scratch_types=[
        pltpu.VMEM_SHARED((NPAD,), jnp.float32),
        pltpu.VMEM((NB, 1, EB), jnp.int32),
        pltpu.VMEM((RPT,), jnp.float32),
        pltpu.VMEM((128,), jnp.float32),
    ],
)
def _deg_kernel(dst_hbm, o0_hbm, o1_hbm, acc, dst_v, zeros_v, ones_v):
    c = lax.axis_index("c")
    s = lax.axis_index("s")
    wid = c * NS + s
    for i in range(RPT // 16):
        zeros_v[pl.ds(i * 16, 16)] = jnp.zeros((16,), jnp.float32)
    for i in range(128 // 16):
        ones_v[pl.ds(i * 16, 16)] = jnp.ones((16,), jnp.float32)
    pltpu.sync_copy(zeros_v, acc.at[pl.ds(s * RPT, RPT)])
    pltpu.sync_copy(dst_hbm.at[wid], dst_v)
    plsc.subcore_barrier()

    @pl.loop(0, NB)
    def _(j):
        pltpu.sync_copy(ones_v.at[pl.ds(0, EB)], acc.at[dst_v.at[j, 0]],
                        add=True)

    plsc.subcore_barrier()

    @pl.when(c == 0)
    def _():
        pltpu.sync_copy(acc.at[pl.ds(s * RPT, RPT)],
                        o0_hbm.at[pl.ds(s * RPT, RPT)])

    @pl.when(c == 1)
    def _():
        pltpu.sync_copy(acc.at[pl.ds(s * RPT, RPT)],
                        o1_hbm.at[pl.ds(s * RPT, RPT)])


@functools.partial(
    pl.kernel,
    out_type=jax.ShapeDtypeStruct((NC, NPAD, D), jnp.float32),
    mesh=_mesh,
    scratch_types=[
        pltpu.VMEM_SHARED((NPAD, D), jnp.float32),
        pltpu.VMEM((HNB, 1, EB), jnp.int32),
        pltpu.VMEM((HNB, 1, EB), jnp.int32),
        pltpu.VMEM((NBUF, EB, D), jnp.float32),
    ] + [pltpu.SemaphoreType.DMA] * NBUF,
)
def _agg_kernel(hp_hbm, src_hbm, dst_hbm, out_hbm, acc, src_v, dst_v, buf,
                *gsems):
    c = lax.axis_index("c")
    s = lax.axis_index("s")
    wid = c * NS + s
    pltpu.sync_copy(hp_hbm.at[pl.ds(s * RPT, RPT)],
                    acc.at[pl.ds(s * RPT, RPT)])
    plsc.subcore_barrier()

    @pl.loop(0, NPH)
    def _(p):
        pltpu.sync_copy(src_hbm.at[wid, pl.ds(p * HNB, HNB)], src_v)
        pltpu.sync_copy(dst_hbm.at[wid, pl.ds(p * HNB, HNB)], dst_v)
        for b in range(NBUF):
            pltpu.async_copy(hp_hbm.at[src_v.at[b, 0]], buf.at[b], gsems[b])

        @pl.loop(0, HNB - 1, step=NBUF)
        def _(j):
            for b in range(NBUF):
                pltpu.make_async_copy(hp_hbm.at[src_v.at[0, 0]], buf.at[b],
                                      gsems[b]).wait()
                pltpu.sync_copy(buf.at[b], acc.at[dst_v.at[j + b, 0]],
                                add=True)

                @pl.when(j + NBUF + b < HNB)
                def _(b=b):
                    pltpu.async_copy(hp_hbm.at[src_v.at[j + NBUF + b, 0]],
                                     buf.at[b], gsems[b])

        pltpu.make_async_copy(hp_hbm.at[src_v.at[0, 0]], buf.at[0],
                              gsems[0]).wait()
        pltpu.sync_copy(buf.at[0], acc.at[dst_v.at[HNB - 1, 0]], add=True)

    plsc.subcore_barrier()
    pltpu.sync_copy(acc.at[pl.ds(s * RPT, RPT)],
                    out_hbm.at[c, pl.ds(s * RPT, RPT)])


def _proj_body(deg1_ref, h_ref, w_ref, o_ref):
    dinv = lax.rsqrt(deg1_ref[...])
    o_ref[...] = jnp.dot(h_ref[...], w_ref[...],
                         preferred_element_type=jnp.float32) * dinv


_proj = pl.pallas_call(
    _proj_body,
    out_shape=jax.ShapeDtypeStruct((NPAD, D), jnp.float32),
)


def _readout_body(deg1_ref, p0_ref, p1_ref, hp_ref, b_ref, wk_ref, bk_ref,
                  wq_ref, bq_ref, s_ref, st_ref, out_ref, r_ref):
    dinv = lax.rsqrt(deg1_ref[...])
    out = (p0_ref[...] + p1_ref[...] - hp_ref[...]) * dinv + b_ref[...]
    out_ref[...] = out
    k = jnp.dot(out, wk_ref[...], preferred_element_type=jnp.float32) + bk_ref[...]
    m = jnp.dot(s_ref[...], out, preferred_element_type=jnp.float32) * (1.0 / (N // G))
    q = jnp.dot(m, wq_ref[...], preferred_element_type=jnp.float32) + bq_ref[...]
    qb = jnp.dot(st_ref[...], q, preferred_element_type=jnp.float32)
    t = jnp.sum(k * qb, axis=1, keepdims=True) * (1.0 / math.sqrt(D))
    att = jax.nn.sigmoid(t)
    r_ref[...] = jnp.dot(s_ref[...], out * att,
                         preferred_element_type=jnp.float32) * (1.0 / (N // G))


_readout = pl.pallas_call(
    _readout_body,
    out_shape=[
        jax.ShapeDtypeStruct((NPAD, D), jnp.float32),
        jax.ShapeDtypeStruct((G, D), jnp.float32),
    ],
)


def kernel(x, edge_index, batch, params):
    src = edge_index[0].reshape(NW, NB, 1, EB)
    dst = edge_index[1].reshape(NW, NB, 1, EB)
    d0, d1 = _deg_kernel(dst)
    deg1 = (d0 + d1 + 1.0).reshape(NPAD, 1)
    xp = jnp.concatenate(
        [x, jnp.zeros((NPAD - N, D), jnp.float32)], axis=0)
    batchp = jnp.concatenate(
        [batch, jnp.full((NPAD - N,), -1, batch.dtype)])
    gids = jnp.arange(G, dtype=batch.dtype)
    s_mat = (batchp[None, :] == gids[:, None]).astype(jnp.float32)
    st_mat = (batchp[:, None] == gids[None, :]).astype(jnp.float32)
    h = xp
    outs = []
    for i in range(L):
        w = params['W%d' % i]
        b = params['b%d' % i].reshape(1, D)
        wq = params['Wq%d' % i]
        bq = params['bq%d' % i].reshape(1, D)
        wk = params['Wk%d' % i]
        bk = params['bk%d' % i].reshape(1, D)
        hp = _proj(deg1, h, w)
        parts = _agg_kernel(hp, src, dst)
        h, r = _readout(deg1, parts[0], parts[1], hp, b, wk, bk, wq, bq,
                        s_mat, st_mat)
        outs.append(r)
    return jnp.concatenate(outs, axis=1)


# speedup vs baseline: 1.0388x; 1.0191x over previous
import functools
import math

import jax
import jax.numpy as jnp
from jax import lax
from jax.experimental import pallas as pl
from jax.experimental.pallas import tpu as pltpu
from jax.experimental.pallas import tpu_sc as plsc

N = 10000
D = 128
E = 320000
G = 100
L = 3

NC = 2
NS = 16
NW = NC * NS
EPW = E // NW
EB = 100
NB = EPW // EB
NPH = 4
HNB = NB // NPH
NBUF = 3
NPAD = 10240
RPT = NPAD // NS

_mesh = plsc.VectorSubcoreMesh(core_axis_name="c", subcore_axis_name="s")


@functools.partial(
    pl.kernel,
    out_type=[
        jax.ShapeDtypeStruct((NPAD,), jnp.float32),
        jax.ShapeDtypeStruct((NPAD,), jnp.float32),
    ],
    mesh=_mesh,
    scratch_types=[
        pltpu.VMEM_SHARED((NPAD,), jnp.float32),
        pltpu.VMEM((NB, 1, EB), jnp.int32),
        pltpu.VMEM((RPT,), jnp.float32),
        pltpu.VMEM((128,), jnp.float32),
    ],
)
def _deg_kernel(dst_hbm, o0_hbm, o1_hbm, acc, dst_v, zeros_v, ones_v):
    c = lax.axis_index("c")
    s = lax.axis_index("s")
    wid = c * NS + s
    for i in range(RPT // 16):
        zeros_v[pl.ds(i * 16, 16)] = jnp.zeros((16,), jnp.float32)
    for i in range(128 // 16):
        ones_v[pl.ds(i * 16, 16)] = jnp.ones((16,), jnp.float32)
    pltpu.sync_copy(zeros_v, acc.at[pl.ds(s * RPT, RPT)])
    pltpu.sync_copy(dst_hbm.at[wid], dst_v)
    plsc.subcore_barrier()

    @pl.loop(0, NB)
    def _(j):
        pltpu.sync_copy(ones_v.at[pl.ds(0, EB)], acc.at[dst_v.at[j, 0]],
                        add=True)

    plsc.subcore_barrier()

    @pl.when(c == 0)
    def _():
        pltpu.sync_copy(acc.at[pl.ds(s * RPT, RPT)],
                        o0_hbm.at[pl.ds(s * RPT, RPT)])

    @pl.when(c == 1)
    def _():
        pltpu.sync_copy(acc.at[pl.ds(s * RPT, RPT)],
                        o1_hbm.at[pl.ds(s * RPT, RPT)])


@functools.partial(
    pl.kernel,
    out_type=jax.ShapeDtypeStruct((NC, NPAD, D), jnp.float32),
    mesh=_mesh,
    scratch_types=[
        pltpu.VMEM_SHARED((NPAD, D), jnp.float32),
        pltpu.VMEM((HNB, 1, EB), jnp.int32),
        pltpu.VMEM((HNB, 1, EB), jnp.int32),
        pltpu.VMEM((NBUF, EB, D), jnp.float32),
    ] + [pltpu.SemaphoreType.DMA] * NBUF,
)
def _agg_kernel(hp_hbm, src_hbm, dst_hbm, out_hbm, acc, src_v, dst_v, buf,
                *gsems):
    c = lax.axis_index("c")
    s = lax.axis_index("s")
    wid = c * NS + s
    pltpu.sync_copy(hp_hbm.at[pl.ds(s * RPT, RPT)],
                    acc.at[pl.ds(s * RPT, RPT)])
    plsc.subcore_barrier()

    @pl.loop(0, NPH)
    def _(p):
        pltpu.sync_copy(src_hbm.at[wid, pl.ds(p * HNB, HNB)], src_v)
        pltpu.sync_copy(dst_hbm.at[wid, pl.ds(p * HNB, HNB)], dst_v)
        for b in range(NBUF):
            pltpu.async_copy(hp_hbm.at[src_v.at[b, 0]], buf.at[b], gsems[b])

        @pl.loop(0, HNB - 1, step=NBUF)
        def _(j):
            for b in range(NBUF):
                pltpu.make_async_copy(hp_hbm.at[src_v.at[0, 0]], buf.at[b],
                                      gsems[b]).wait()
                pltpu.sync_copy(buf.at[b], acc.at[dst_v.at[j + b, 0]],
                                add=True)

                @pl.when(j + NBUF + b < HNB)
                def _(b=b):
                    pltpu.async_copy(hp_hbm.at[src_v.at[j + NBUF + b, 0]],
                                     buf.at[b], gsems[b])

        pltpu.make_async_copy(hp_hbm.at[src_v.at[0, 0]], buf.at[0],
                              gsems[0]).wait()
        pltpu.sync_copy(buf.at[0], acc.at[dst_v.at[HNB - 1, 0]], add=True)

    plsc.subcore_barrier()
    pltpu.sync_copy(acc.at[pl.ds(s * RPT, RPT)],
                    out_hbm.at[c, pl.ds(s * RPT, RPT)])


def _proj_body(deg1_ref, h_ref, w_ref, o_ref):
    dinv = lax.rsqrt(deg1_ref[...])
    o_ref[...] = jnp.dot(h_ref[...], w_ref[...],
                         preferred_element_type=jnp.float32) * dinv


_proj = pl.pallas_call(
    _proj_body,
    out_shape=jax.ShapeDtypeStruct((NPAD, D), jnp.float32),
)


def _epi_proj_body(deg1_ref, p0_ref, p1_ref, hp_ref, b_ref, wn_ref,
                   out_ref, hpn_ref):
    dinv = lax.rsqrt(deg1_ref[...])
    out = (p0_ref[...] + p1_ref[...] - hp_ref[...]) * dinv + b_ref[...]
    out_ref[...] = out
    hpn_ref[...] = jnp.dot(out, wn_ref[...],
                           preferred_element_type=jnp.float32) * dinv


_epi_proj = pl.pallas_call(
    _epi_proj_body,
    out_shape=[
        jax.ShapeDtypeStruct((NPAD, D), jnp.float32),
        jax.ShapeDtypeStruct((NPAD, D), jnp.float32),
    ],
)


def _epi_last_body(deg1_ref, p0_ref, p1_ref, hp_ref, b_ref, out_ref):
    dinv = lax.rsqrt(deg1_ref[...])
    out_ref[...] = (p0_ref[...] + p1_ref[...] - hp_ref[...]) * dinv + b_ref[...]


_epi_last = pl.pallas_call(
    _epi_last_body,
    out_shape=jax.ShapeDtypeStruct((NPAD, D), jnp.float32),
)


def _readout_body(out_ref, wk_ref, bk_ref, wq_ref, bq_ref, s_ref, st_ref,
                  r_ref):
    out = out_ref[...]
    k = jnp.dot(out, wk_ref[...], preferred_element_type=jnp.float32) + bk_ref[...]
    m = jnp.dot(s_ref[...], out, preferred_element_type=jnp.float32) * (1.0 / (N // G))
    q = jnp.dot(m, wq_ref[...], preferred_element_type=jnp.float32) + bq_ref[...]
    qb = jnp.dot(st_ref[...], q, preferred_element_type=jnp.float32)
    t = jnp.sum(k * qb, axis=1, keepdims=True) * (1.0 / math.sqrt(D))
    att = jax.nn.sigmoid(t)
    r_ref[...] = jnp.dot(s_ref[...], out * att,
                         preferred_element_type=jnp.float32) * (1.0 / (N // G))


_readout = pl.pallas_call(
    _readout_body,
    out_shape=jax.ShapeDtypeStruct((G, D), jnp.float32),
)


def kernel(x, edge_index, batch, params):
    src = edge_index[0].reshape(NW, NB, 1, EB)
    dst = edge_index[1].reshape(NW, NB, 1, EB)
    d0, d1 = _deg_kernel(dst)
    deg1 = (d0 + d1 + 1.0).reshape(NPAD, 1)
    xp = jnp.concatenate(
        [x, jnp.zeros((NPAD - N, D), jnp.float32)], axis=0)
    batchp = jnp.concatenate(
        [batch, jnp.full((NPAD - N,), -1, batch.dtype)])
    gids = jnp.arange(G, dtype=batch.dtype)
    s_mat = (batchp[None, :] == gids[:, None]).astype(jnp.float32)
    st_mat = (batchp[:, None] == gids[None, :]).astype(jnp.float32)
    hp = _proj(deg1, xp, params['W0'])
    outs = []
    for i in range(L):
        b = params['b%d' % i].reshape(1, D)
        parts = _agg_kernel(hp, src, dst)
        if i + 1 < L:
            out, hp = _epi_proj(deg1, parts[0], parts[1], hp, b,
                                params['W%d' % (i + 1)])
        else:
            out = _epi_last(deg1, parts[0], parts[1], hp, b)
        r = _readout(out, params['Wk%d' % i], params['bk%d' % i].reshape(1, D),
                     params['Wq%d' % i], params['bq%d' % i].reshape(1, D),
                     s_mat, st_mat)
        outs.append(r)
    return jnp.concatenate(outs, axis=1)


# speedup vs baseline: 1.0658x; 1.0261x over previous
import functools
import math

import jax
import jax.numpy as jnp
from jax import lax
from jax.experimental import pallas as pl
from jax.experimental.pallas import tpu as pltpu
from jax.experimental.pallas import tpu_sc as plsc

N = 10000
D = 128
E = 320000
G = 100
L = 3

NC = 2
NS = 16
NW = NC * NS
EPW = E // NW
EB = 100
NB = EPW // EB
NPH = 4
HNB = NB // NPH
NBUF = 3
NPAD = 10240
RPT = NPAD // NS

_mesh = plsc.VectorSubcoreMesh(core_axis_name="c", subcore_axis_name="s")


@functools.partial(
    pl.kernel,
    out_type=[
        jax.ShapeDtypeStruct((NPAD,), jnp.float32),
        jax.ShapeDtypeStruct((NPAD,), jnp.float32),
    ],
    mesh=_mesh,
    scratch_types=[
        pltpu.VMEM_SHARED((NPAD,), jnp.float32),
        pltpu.VMEM((NB, 1, EB), jnp.int32),
        pltpu.VMEM((RPT,), jnp.float32),
        pltpu.VMEM((128,), jnp.float32),
    ],
)
def _deg_kernel(dst_hbm, o0_hbm, o1_hbm, acc, dst_v, zeros_v, ones_v):
    c = lax.axis_index("c")
    s = lax.axis_index("s")
    wid = c * NS + s
    for i in range(RPT // 16):
        zeros_v[pl.ds(i * 16, 16)] = jnp.zeros((16,), jnp.float32)
    for i in range(128 // 16):
        ones_v[pl.ds(i * 16, 16)] = jnp.ones((16,), jnp.float32)
    pltpu.sync_copy(zeros_v, acc.at[pl.ds(s * RPT, RPT)])
    pltpu.sync_copy(dst_hbm.at[wid], dst_v)
    plsc.subcore_barrier()

    @pl.loop(0, NB)
    def _(j):
        pltpu.sync_copy(ones_v.at[pl.ds(0, EB)], acc.at[dst_v.at[j, 0]],
                        add=True)

    plsc.subcore_barrier()

    @pl.when(c == 0)
    def _():
        pltpu.sync_copy(acc.at[pl.ds(s * RPT, RPT)],
                        o0_hbm.at[pl.ds(s * RPT, RPT)])

    @pl.when(c == 1)
    def _():
        pltpu.sync_copy(acc.at[pl.ds(s * RPT, RPT)],
                        o1_hbm.at[pl.ds(s * RPT, RPT)])


@functools.partial(
    pl.kernel,
    out_type=jax.ShapeDtypeStruct((NC, NPAD, D), jnp.float32),
    mesh=_mesh,
    scratch_types=[
        pltpu.VMEM_SHARED((NPAD, D), jnp.float32),
        pltpu.VMEM((HNB, 1, EB), jnp.int32),
        pltpu.VMEM((HNB, 1, EB), jnp.int32),
        pltpu.VMEM((NBUF, EB, D), jnp.float32),
    ] + [pltpu.SemaphoreType.DMA] * NBUF,
)
def _agg_kernel(hp_hbm, src_hbm, dst_hbm, out_hbm, acc, src_v, dst_v, buf,
                *gsems):
    c = lax.axis_index("c")
    s = lax.axis_index("s")
    wid = c * NS + s
    pltpu.sync_copy(src_hbm.at[wid, pl.ds(0, HNB)], src_v)
    pltpu.sync_copy(dst_hbm.at[wid, pl.ds(0, HNB)], dst_v)
    for b in range(NBUF):
        pltpu.async_copy(hp_hbm.at[src_v.at[b, 0]], buf.at[b], gsems[b])
    pltpu.sync_copy(hp_hbm.at[pl.ds(s * RPT, RPT)],
                    acc.at[pl.ds(s * RPT, RPT)])
    plsc.subcore_barrier()

    @pl.loop(0, NPH)
    def _(p):
        @pl.when(p > 0)
        def _():
            pltpu.sync_copy(src_hbm.at[wid, pl.ds(p * HNB, HNB)], src_v)
            pltpu.sync_copy(dst_hbm.at[wid, pl.ds(p * HNB, HNB)], dst_v)
            for b in range(NBUF):
                pltpu.async_copy(hp_hbm.at[src_v.at[b, 0]], buf.at[b],
                                 gsems[b])

        @pl.loop(0, HNB - 1, step=NBUF)
        def _(j):
            for b in range(NBUF):
                pltpu.make_async_copy(hp_hbm.at[src_v.at[0, 0]], buf.at[b],
                                      gsems[b]).wait()
                pltpu.sync_copy(buf.at[b], acc.at[dst_v.at[j + b, 0]],
                                add=True)

                @pl.when(j + NBUF + b < HNB)
                def _(b=b):
                    pltpu.async_copy(hp_hbm.at[src_v.at[j + NBUF + b, 0]],
                                     buf.at[b], gsems[b])

        pltpu.make_async_copy(hp_hbm.at[src_v.at[0, 0]], buf.at[0],
                              gsems[0]).wait()
        pltpu.sync_copy(buf.at[0], acc.at[dst_v.at[HNB - 1, 0]], add=True)

    plsc.subcore_barrier()
    pltpu.sync_copy(acc.at[pl.ds(s * RPT, RPT)],
                    out_hbm.at[c, pl.ds(s * RPT, RPT)])


def _proj_body(deg1_ref, h_ref, w_ref, o_ref):
    dinv = lax.rsqrt(deg1_ref[...])
    o_ref[...] = jnp.dot(h_ref[...], w_ref[...],
                         preferred_element_type=jnp.float32) * dinv


_proj = pl.pallas_call(
    _proj_body,
    out_shape=jax.ShapeDtypeStruct((NPAD, D), jnp.float32),
)


def _hpn_body(deg1_ref, p0_ref, p1_ref, hp_ref, b_ref, wn_ref, hpn_ref):
    dinv = lax.rsqrt(deg1_ref[...])
    out = (p0_ref[...] + p1_ref[...] - hp_ref[...]) * dinv + b_ref[...]
    hpn_ref[...] = jnp.dot(out, wn_ref[...],
                           preferred_element_type=jnp.float32) * dinv


_hpn = pl.pallas_call(
    _hpn_body,
    out_shape=jax.ShapeDtypeStruct((NPAD, D), jnp.float32),
)


def _readout_body(deg1_ref, p0_ref, p1_ref, hp_ref, b_ref, wk_ref, bk_ref,
                  wq_ref, bq_ref, s_ref, st_ref, r_ref):
    dinv = lax.rsqrt(deg1_ref[...])
    out = (p0_ref[...] + p1_ref[...] - hp_ref[...]) * dinv + b_ref[...]
    k = jnp.dot(out, wk_ref[...], preferred_element_type=jnp.float32) + bk_ref[...]
    m = jnp.dot(s_ref[...], out, preferred_element_type=jnp.float32) * (1.0 / (N // G))
    q = jnp.dot(m, wq_ref[...], preferred_element_type=jnp.float32) + bq_ref[...]
    qb = jnp.dot(st_ref[...], q, preferred_element_type=jnp.float32)
    t = jnp.sum(k * qb, axis=1, keepdims=True) * (1.0 / math.sqrt(D))
    att = jax.nn.sigmoid(t)
    r_ref[...] = jnp.dot(s_ref[...], out * att,
                         preferred_element_type=jnp.float32) * (1.0 / (N // G))


_readout = pl.pallas_call(
    _readout_body,
    out_shape=jax.ShapeDtypeStruct((G, D), jnp.float32),
)


def kernel(x, edge_index, batch, params):
    src = edge_index[0].reshape(NW, NB, 1, EB)
    dst = edge_index[1].reshape(NW, NB, 1, EB)
    d0, d1 = _deg_kernel(dst)
    deg1 = (d0 + d1 + 1.0).reshape(NPAD, 1)
    xp = jnp.concatenate(
        [x, jnp.zeros((NPAD - N, D), jnp.float32)], axis=0)
    batchp = jnp.concatenate(
        [batch, jnp.full((NPAD - N,), -1, batch.dtype)])
    gids = jnp.arange(G, dtype=batch.dtype)
    s_mat = (batchp[None, :] == gids[:, None]).astype(jnp.float32)
    st_mat = (batchp[:, None] == gids[None, :]).astype(jnp.float32)
    hp = _proj(deg1, xp, params['W0'])
    outs = []
    for i in range(L):
        b = params['b%d' % i].reshape(1, D)
        parts = _agg_kernel(hp, src, dst)
        hp_cur = hp
        if i + 1 < L:
            hp = _hpn(deg1, parts[0], parts[1], hp_cur, b,
                      params['W%d' % (i + 1)])
        r = _readout(deg1, parts[0], parts[1], hp_cur, b,
                     params['Wk%d' % i], params['bk%d' % i].reshape(1, D),
                     params['Wq%d' % i], params['bq%d' % i].reshape(1, D),
                     s_mat, st_mat)
        outs.append(r)
    return jnp.concatenate(outs, axis=1)


# speedup vs baseline: 1.1308x; 1.0610x over previous
import functools
import math

import jax
import jax.numpy as jnp
from jax import lax
from jax.experimental import pallas as pl
from jax.experimental.pallas import tpu as pltpu
from jax.experimental.pallas import tpu_sc as plsc

N = 10000
D = 128
E = 320000
G = 100
L = 3

NC = 2
NS = 16
NW = NC * NS
EPW = E // NW
EB = 80
NB = EPW // EB
NBUF = 3
NPAD = 10240
RPT = NPAD // NS

_mesh = plsc.VectorSubcoreMesh(core_axis_name="c", subcore_axis_name="s")


@functools.partial(
    pl.kernel,
    out_type=[
        jax.ShapeDtypeStruct((NPAD,), jnp.float32),
        jax.ShapeDtypeStruct((NPAD,), jnp.float32),
    ],
    mesh=_mesh,
    scratch_types=[
        pltpu.VMEM_SHARED((NPAD,), jnp.float32),
        pltpu.VMEM((NB, 1, EB), jnp.int32),
        pltpu.VMEM((RPT,), jnp.float32),
        pltpu.VMEM((128,), jnp.float32),
    ],
)
def _deg_kernel(dst_hbm, o0_hbm, o1_hbm, acc, dst_v, zeros_v, ones_v):
    c = lax.axis_index("c")
    s = lax.axis_index("s")
    wid = c * NS + s
    for i in range(RPT // 16):
        zeros_v[pl.ds(i * 16, 16)] = jnp.zeros((16,), jnp.float32)
    for i in range(128 // 16):
        ones_v[pl.ds(i * 16, 16)] = jnp.ones((16,), jnp.float32)
    pltpu.sync_copy(zeros_v, acc.at[pl.ds(s * RPT, RPT)])
    pltpu.sync_copy(dst_hbm.at[wid], dst_v)
    plsc.subcore_barrier()

    @pl.loop(0, NB)
    def _(j):
        pltpu.sync_copy(ones_v.at[pl.ds(0, EB)], acc.at[dst_v.at[j, 0]],
                        add=True)

    plsc.subcore_barrier()

    @pl.when(c == 0)
    def _():
        pltpu.sync_copy(acc.at[pl.ds(s * RPT, RPT)],
                        o0_hbm.at[pl.ds(s * RPT, RPT)])

    @pl.when(c == 1)
    def _():
        pltpu.sync_copy(acc.at[pl.ds(s * RPT, RPT)],
                        o1_hbm.at[pl.ds(s * RPT, RPT)])


@functools.partial(
    pl.kernel,
    out_type=jax.ShapeDtypeStruct((NC, NPAD, D), jnp.float32),
    mesh=_mesh,
    scratch_types=[
        pltpu.VMEM_SHARED((NPAD, D), jnp.float32),
        pltpu.VMEM((NB, 1, EB), jnp.int32),
        pltpu.VMEM((NBUF, 1, EB), jnp.int32),
        pltpu.VMEM((NBUF, 1, EB), jnp.int32),
        pltpu.VMEM((NBUF, EB, D), jnp.float32),
    ] + [pltpu.SemaphoreType.DMA] * NBUF,
)
def _agg_kernel(hp_hbm, pk_hbm, out_hbm, acc, pk_v, sidx, didx, buf, *gsems):
    c = lax.axis_index("c")
    s = lax.axis_index("s")
    wid = c * NS + s

    def unpack(j, b):
        for k in range(EB // 16):
            v = pk_v[j, 0, pl.ds(k * 16, 16)]
            sidx[b, 0, pl.ds(k * 16, 16)] = v & 0xFFFF
            didx[b, 0, pl.ds(k * 16, 16)] = lax.shift_right_logical(v, 16)

    pltpu.sync_copy(pk_hbm.at[wid], pk_v)
    for b in range(NBUF):
        unpack(b, b)
        pltpu.async_copy(hp_hbm.at[sidx.at[b, 0]], buf.at[b], gsems[b])
    pltpu.sync_copy(hp_hbm.at[pl.ds(s * RPT, RPT)],
                    acc.at[pl.ds(s * RPT, RPT)])
    plsc.subcore_barrier()

    @pl.loop(0, NB - 2, step=NBUF)
    def _(j):
        for b in range(NBUF):
            pltpu.make_async_copy(hp_hbm.at[sidx.at[b, 0]], buf.at[b],
                                  gsems[b]).wait()
            pltpu.sync_copy(buf.at[b], acc.at[didx.at[b, 0]], add=True)

            @pl.when(j + NBUF + b < NB)
            def _(b=b):
                unpack(j + NBUF + b, b)
                pltpu.async_copy(hp_hbm.at[sidx.at[b, 0]], buf.at[b],
                                 gsems[b])

    for b, _jj in ((0, NB - 2), (1, NB - 1)):
        pltpu.make_async_copy(hp_hbm.at[sidx.at[b, 0]], buf.at[b],
                              gsems[b]).wait()
        pltpu.sync_copy(buf.at[b], acc.at[didx.at[b, 0]], add=True)

    plsc.subcore_barrier()
    pltpu.sync_copy(acc.at[pl.ds(s * RPT, RPT)],
                    out_hbm.at[c, pl.ds(s * RPT, RPT)])


def _proj_body(deg1_ref, h_ref, w_ref, o_ref):
    dinv = lax.rsqrt(deg1_ref[...])
    o_ref[...] = jnp.dot(h_ref[...], w_ref[...],
                         preferred_element_type=jnp.float32) * dinv


_proj = pl.pallas_call(
    _proj_body,
    out_shape=jax.ShapeDtypeStruct((NPAD, D), jnp.float32),
)


def _hpn_body(deg1_ref, p0_ref, p1_ref, hp_ref, b_ref, wn_ref, hpn_ref):
    dinv = lax.rsqrt(deg1_ref[...])
    out = (p0_ref[...] + p1_ref[...] - hp_ref[...]) * dinv + b_ref[...]
    hpn_ref[...] = jnp.dot(out, wn_ref[...],
                           preferred_element_type=jnp.float32) * dinv


_hpn = pl.pallas_call(
    _hpn_body,
    out_shape=jax.ShapeDtypeStruct((NPAD, D), jnp.float32),
)


def _readout_body(deg1_ref, p0_ref, p1_ref, hp_ref, b_ref, wk_ref, bk_ref,
                  wq_ref, bq_ref, s_ref, st_ref, r_ref):
    dinv = lax.rsqrt(deg1_ref[...])
    out = (p0_ref[...] + p1_ref[...] - hp_ref[...]) * dinv + b_ref[...]
    k = jnp.dot(out, wk_ref[...], preferred_element_type=jnp.float32) + bk_ref[...]
    m = jnp.dot(s_ref[...], out, preferred_element_type=jnp.float32) * (1.0 / (N // G))
    q = jnp.dot(m, wq_ref[...], preferred_element_type=jnp.float32) + bq_ref[...]
    qb = jnp.dot(st_ref[...], q, preferred_element_type=jnp.float32)
    t = jnp.sum(k * qb, axis=1, keepdims=True) * (1.0 / math.sqrt(D))
    att = jax.nn.sigmoid(t)
    r_ref[...] = jnp.dot(s_ref[...], out * att,
                         preferred_element_type=jnp.float32) * (1.0 / (N // G))


_readout = pl.pallas_call(
    _readout_body,
    out_shape=jax.ShapeDtypeStruct((G, D), jnp.float32),
)


def kernel(x, edge_index, batch, params):
    src = edge_index[0].reshape(NW, NB, 1, EB)
    dst = edge_index[1].reshape(NW, NB, 1, EB)
    pk = src | (dst << 16)
    d0, d1 = _deg_kernel(dst)
    deg1 = (d0 + d1 + 1.0).reshape(NPAD, 1)
    xp = jnp.concatenate(
        [x, jnp.zeros((NPAD - N, D), jnp.float32)], axis=0)
    batchp = jnp.concatenate(
        [batch, jnp.full((NPAD - N,), -1, batch.dtype)])
    gids = jnp.arange(G, dtype=batch.dtype)
    s_mat = (batchp[None, :] == gids[:, None]).astype(jnp.float32)
    st_mat = (batchp[:, None] == gids[None, :]).astype(jnp.float32)
    hp = _proj(deg1, xp, params['W0'])
    outs = []
    for i in range(L):
        b = params['b%d' % i].reshape(1, D)
        parts = _agg_kernel(hp, pk)
        hp_cur = hp
        if i + 1 < L:
            hp = _hpn(deg1, parts[0], parts[1], hp_cur, b,
                      params['W%d' % (i + 1)])
        r = _readout(deg1, parts[0], parts[1], hp_cur, b,
                     params['Wk%d' % i], params['bk%d' % i].reshape(1, D),
                     params['Wq%d' % i], params['bq%d' % i].reshape(1, D),
                     s_mat, st_mat)
        outs.append(r)
    return jnp.concatenate(outs, axis=1)


# speedup vs baseline: 1.1335x; 1.0024x over previous
import functools
import math

import jax
import jax.numpy as jnp
from jax import lax
from jax.experimental import pallas as pl
from jax.experimental.pallas import tpu as pltpu
from jax.experimental.pallas import tpu_sc as plsc

N = 10000
D = 128
E = 320000
G = 100
L = 3

NC = 2
NS = 16
NW = NC * NS
EPW = E // NW
EB = 80
NB = EPW // EB
NBUF = 3
NPAD = 10240
RPT = NPAD // NS

_mesh = plsc.VectorSubcoreMesh(core_axis_name="c", subcore_axis_name="s")


@functools.partial(
    pl.kernel,
    out_type=[
        jax.ShapeDtypeStruct((NPAD,), jnp.float32),
        jax.ShapeDtypeStruct((NPAD,), jnp.float32),
    ],
    mesh=_mesh,
    scratch_types=[
        pltpu.VMEM_SHARED((NPAD,), jnp.float32),
        pltpu.VMEM((NB, 1, EB), jnp.int32),
        pltpu.VMEM((RPT,), jnp.float32),
        pltpu.VMEM((128,), jnp.float32),
        pltpu.SemaphoreType.DMA,
    ],
)
def _deg_kernel(dst_hbm, o0_hbm, o1_hbm, acc, dst_v, zeros_v, ones_v, sem):
    c = lax.axis_index("c")
    s = lax.axis_index("s")
    wid = c * NS + s
    for i in range(RPT // 16):
        zeros_v[pl.ds(i * 16, 16)] = jnp.zeros((16,), jnp.float32)
    for i in range(128 // 16):
        ones_v[pl.ds(i * 16, 16)] = jnp.ones((16,), jnp.float32)
    pltpu.sync_copy(zeros_v, acc.at[pl.ds(s * RPT, RPT)])
    pltpu.sync_copy(dst_hbm.at[wid], dst_v)
    plsc.subcore_barrier()

    @pl.loop(0, NB, step=5)
    def _(j):
        for t in range(5):
            pltpu.async_copy(ones_v.at[pl.ds(0, EB)],
                             acc.at[dst_v.at[j + t, 0]], sem, add=True)
        for t in range(5):
            pltpu.make_async_copy(ones_v.at[pl.ds(0, EB)],
                                  acc.at[dst_v.at[0, 0]], sem).wait()

    plsc.subcore_barrier()

    @pl.when(c == 0)
    def _():
        pltpu.sync_copy(acc.at[pl.ds(s * RPT, RPT)],
                        o0_hbm.at[pl.ds(s * RPT, RPT)])

    @pl.when(c == 1)
    def _():
        pltpu.sync_copy(acc.at[pl.ds(s * RPT, RPT)],
                        o1_hbm.at[pl.ds(s * RPT, RPT)])


@functools.partial(
    pl.kernel,
    out_type=jax.ShapeDtypeStruct((NC, NPAD, D), jnp.float32),
    mesh=_mesh,
    scratch_types=[
        pltpu.VMEM_SHARED((NPAD, D), jnp.float32),
        pltpu.VMEM((NB, 1, EB), jnp.int32),
        pltpu.VMEM((NBUF, 1, EB), jnp.int32),
        pltpu.VMEM((NBUF, 1, EB), jnp.int32),
        pltpu.VMEM((NBUF, EB, D), jnp.float32),
    ] + [pltpu.SemaphoreType.DMA] * NBUF,
)
def _agg_kernel(hp_hbm, pk_hbm, out_hbm, acc, pk_v, sidx, didx, buf, *gsems):
    c = lax.axis_index("c")
    s = lax.axis_index("s")
    wid = c * NS + s

    def unpack(j, b):
        for k in range(EB // 16):
            v = pk_v[j, 0, pl.ds(k * 16, 16)]
            sidx[b, 0, pl.ds(k * 16, 16)] = v & 0xFFFF
            didx[b, 0, pl.ds(k * 16, 16)] = lax.shift_right_logical(v, 16)

    pltpu.sync_copy(pk_hbm.at[wid], pk_v)
    for b in range(NBUF):
        unpack(b, b)
        pltpu.async_copy(hp_hbm.at[sidx.at[b, 0]], buf.at[b], gsems[b])
    pltpu.sync_copy(hp_hbm.at[pl.ds(s * RPT, RPT)],
                    acc.at[pl.ds(s * RPT, RPT)])
    plsc.subcore_barrier()

    @pl.loop(0, NB - 2, step=NBUF)
    def _(j):
        for b in range(NBUF):
            pltpu.make_async_copy(hp_hbm.at[sidx.at[b, 0]], buf.at[b],
                                  gsems[b]).wait()
            pltpu.sync_copy(buf.at[b], acc.at[didx.at[b, 0]], add=True)

            @pl.when(j + NBUF + b < NB)
            def _(b=b):
                unpack(j + NBUF + b, b)
                pltpu.async_copy(hp_hbm.at[sidx.at[b, 0]], buf.at[b],
                                 gsems[b])

    for b, _jj in ((0, NB - 2), (1, NB - 1)):
        pltpu.make_async_copy(hp_hbm.at[sidx.at[b, 0]], buf.at[b],
                              gsems[b]).wait()
        pltpu.sync_copy(buf.at[b], acc.at[didx.at[b, 0]], add=True)

    plsc.subcore_barrier()
    pltpu.sync_copy(acc.at[pl.ds(s * RPT, RPT)],
                    out_hbm.at[c, pl.ds(s * RPT, RPT)])


def _proj_body(deg1_ref, h_ref, w_ref, o_ref):
    dinv = lax.rsqrt(deg1_ref[...])
    o_ref[...] = jnp.dot(h_ref[...], w_ref[...],
                         preferred_element_type=jnp.float32) * dinv


_proj = pl.pallas_call(
    _proj_body,
    out_shape=jax.ShapeDtypeStruct((NPAD, D), jnp.float32),
)


def _hpn_body(deg1_ref, p0_ref, p1_ref, hp_ref, b_ref, wn_ref, hpn_ref):
    dinv = lax.rsqrt(deg1_ref[...])
    out = (p0_ref[...] + p1_ref[...] - hp_ref[...]) * dinv + b_ref[...]
    hpn_ref[...] = jnp.dot(out, wn_ref[...],
                           preferred_element_type=jnp.float32) * dinv


_hpn = pl.pallas_call(
    _hpn_body,
    out_shape=jax.ShapeDtypeStruct((NPAD, D), jnp.float32),
)


def _readout_body(deg1_ref, p0_ref, p1_ref, hp_ref, b_ref, wk_ref, bk_ref,
                  wq_ref, bq_ref, s_ref, st_ref, r_ref):
    dinv = lax.rsqrt(deg1_ref[...])
    out = (p0_ref[...] + p1_ref[...] - hp_ref[...]) * dinv + b_ref[...]
    k = jnp.dot(out, wk_ref[...], preferred_element_type=jnp.float32) + bk_ref[...]
    m = jnp.dot(s_ref[...], out, preferred_element_type=jnp.float32) * (1.0 / (N // G))
    q = jnp.dot(m, wq_ref[...], preferred_element_type=jnp.float32) + bq_ref[...]
    qb = jnp.dot(st_ref[...], q, preferred_element_type=jnp.float32)
    t = jnp.sum(k * qb, axis=1, keepdims=True) * (1.0 / math.sqrt(D))
    att = jax.nn.sigmoid(t)
    r_ref[...] = jnp.dot(s_ref[...], out * att,
                         preferred_element_type=jnp.float32) * (1.0 / (N // G))


_readout = pl.pallas_call(
    _readout_body,
    out_shape=jax.ShapeDtypeStruct((G, D), jnp.float32),
)


def kernel(x, edge_index, batch, params):
    src = edge_index[0].reshape(NW, NB, 1, EB)
    dst = edge_index[1].reshape(NW, NB, 1, EB)
    pk = src | (dst << 16)
    d0, d1 = _deg_kernel(dst)
    deg1 = (d0 + d1 + 1.0).reshape(NPAD, 1)
    xp = jnp.concatenate(
        [x, jnp.zeros((NPAD - N, D), jnp.float32)], axis=0)
    batchp = jnp.concatenate(
        [batch, jnp.full((NPAD - N,), -1, batch.dtype)])
    gids = jnp.arange(G, dtype=batch.dtype)
    s_mat = (batchp[None, :] == gids[:, None]).astype(jnp.float32)
    st_mat = (batchp[:, None] == gids[None, :]).astype(jnp.float32)
    hp = _proj(deg1, xp, params['W0'])
    outs = []
    for i in range(L):
        b = params['b%d' % i].reshape(1, D)
        parts = _agg_kernel(hp, pk)
        hp_cur = hp
        if i + 1 < L:
            hp = _hpn(deg1, parts[0], parts[1], hp_cur, b,
                      params['W%d' % (i + 1)])
        r = _readout(deg1, parts[0], parts[1], hp_cur, b,
                     params['Wk%d' % i], params['bk%d' % i].reshape(1, D),
                     params['Wq%d' % i], params['bq%d' % i].reshape(1, D),
                     s_mat, st_mat)
        outs.append(r)
    return jnp.concatenate(outs, axis=1)
